# baseline (speedup 1.0000x reference)
import functools

import jax
import jax.numpy as jnp
from jax import lax
from jax.experimental import pallas as pl
from jax.experimental.pallas import tpu as pltpu
from jax.experimental.pallas import tpu_sc as plsc

_ORDER = 16
_L = 16
_NC = 2
_NS = 16
_NW = _NC * _NS
_ROWS = 200
_COLS = 16384
_CW = _COLS // _NW
_CR = 40
_CHUNK_ROWS = (8, 16, 32, 40, 40, 32, 24, 8)
_CHUNK_OFFS = (0, 8, 24, 56, 96, 136, 168, 192)
_NCHUNK = len(_CHUNK_ROWS)
_CVEC = _CW // _L
_NBUF = 3


def _body(inp_hbm, table_hbm, out_hbm, table_v, ins, outs, sin, sout):
    wid = lax.axis_index("s") * _NC + lax.axis_index("c")
    col0 = wid * _CW

    def start_in(ci):
        b = ci % _NBUF
        r0, nr = _CHUNK_OFFS[ci], _CHUNK_ROWS[ci]
        return pltpu.async_copy(
            inp_hbm.at[pl.ds(r0, nr), pl.ds(col0, _CW)],
            ins.at[b, pl.ds(0, nr)], sin.at[b])

    def start_out(ci):
        b = ci % _NBUF
        r0, nr = _CHUNK_OFFS[ci], _CHUNK_ROWS[ci]
        return pltpu.async_copy(
            outs.at[b, pl.ds(0, nr)],
            out_hbm.at[pl.ds(r0, nr), pl.ds(col0, _CW)], sout.at[b])

    in_copies = {ci: start_in(ci) for ci in range(min(_NBUF, _NCHUNK))}
    out_copies = {}
    pltpu.sync_copy(table_hbm, table_v)
    tbl = table_v[...]
    for ci in range(_NCHUNK):
        b = ci % _NBUF
        in_copies[ci].wait()
        if ci >= _NBUF:
            out_copies[ci - _NBUF].wait()
        iv, ov = ins.at[b], outs.at[b]

        @plsc.parallel_loop(0, _CHUNK_ROWS[ci] * _CVEC, unroll=16)
        def _gather(i, iv=iv, ov=ov):
            r = lax.shift_right_logical(i, 5)
            c = lax.shift_left(lax.bitwise_and(i, _CVEC - 1), 4)
            idx = iv[r, pl.ds(c, _L)]
            ov[r, pl.ds(c, _L)] = jnp.take_along_axis(
                tbl, idx, axis=0, mode="promise_in_bounds")

        out_copies[ci] = start_out(ci)
        if ci + _NBUF < _NCHUNK:
            in_copies[ci + _NBUF] = start_in(ci + _NBUF)

    for ci in range(max(0, _NCHUNK - _NBUF), _NCHUNK):
        out_copies[ci].wait()


def kernel(input, val_table):
    xt = input.T
    mesh = plsc.VectorSubcoreMesh(core_axis_name="c", subcore_axis_name="s")
    run = pl.kernel(
        _body,
        mesh=mesh,
        out_type=jax.ShapeDtypeStruct((_ROWS, _COLS), jnp.float32),
        scratch_types=[
            pltpu.VMEM((_ORDER,), jnp.float32),
            pltpu.VMEM((_NBUF, _CR, _CW), jnp.int32),
            pltpu.VMEM((_NBUF, _CR, _CW), jnp.float32),
            pltpu.SemaphoreType.DMA((_NBUF,)),
            pltpu.SemaphoreType.DMA((_NBUF,)),
        ],
        compiler_params=pltpu.CompilerParams(
            needs_layout_passes=False, use_tc_tiling_on_sc=True,
            skip_device_barrier=True),
    )
    return run(xt, val_table).T


# speedup vs baseline: 1.0379x; 1.0379x over previous
import functools

import jax
import jax.numpy as jnp
from jax import lax
from jax.experimental import pallas as pl
from jax.experimental.pallas import tpu as pltpu
from jax.experimental.pallas import tpu_sc as plsc

_ORDER = 16
_L = 16
_NC = 2
_NS = 16
_NW = _NC * _NS
_ROWS = 200
_COLS = 16384
_CW = _COLS // _NW
_CR = 40
_CHUNK_ROWS = (8, 32, 40, 40, 40, 32, 8)
_CHUNK_OFFS = (0, 8, 40, 80, 120, 160, 192)
_NCHUNK = len(_CHUNK_ROWS)
_CVEC = _CW // _L
_NBUF = 3


def _body(inp_hbm, table_hbm, out_hbm, table_v, ins, outs, sin, sout):
    wid = lax.axis_index("s") * _NC + lax.axis_index("c")
    col0 = wid * _CW

    def start_in(ci):
        b = ci % _NBUF
        r0, nr = _CHUNK_OFFS[ci], _CHUNK_ROWS[ci]
        return pltpu.async_copy(
            inp_hbm.at[pl.ds(r0, nr), pl.ds(col0, _CW)],
            ins.at[b, pl.ds(0, nr)], sin.at[b])

    def start_out(ci):
        b = ci % _NBUF
        r0, nr = _CHUNK_OFFS[ci], _CHUNK_ROWS[ci]
        return pltpu.async_copy(
            outs.at[b, pl.ds(0, nr)],
            out_hbm.at[pl.ds(r0, nr), pl.ds(col0, _CW)], sout.at[b])

    in_copies = {ci: start_in(ci) for ci in range(min(_NBUF, _NCHUNK))}
    out_copies = {}
    pltpu.sync_copy(table_hbm, table_v)
    tbl = table_v[...]
    for ci in range(_NCHUNK):
        b = ci % _NBUF
        in_copies[ci].wait()
        if ci >= _NBUF:
            out_copies[ci - _NBUF].wait()
        iv, ov = ins.at[b], outs.at[b]

        @plsc.parallel_loop(0, _CHUNK_ROWS[ci] * _CVEC, unroll=16)
        def _gather(i, iv=iv, ov=ov):
            r = lax.shift_right_logical(i, 5)
            c = lax.shift_left(lax.bitwise_and(i, _CVEC - 1), 4)
            idx = iv[r, pl.ds(c, _L)]
            ov[r, pl.ds(c, _L)] = jnp.take_along_axis(
                tbl, idx, axis=0, mode="promise_in_bounds")

        out_copies[ci] = start_out(ci)
        if ci + _NBUF < _NCHUNK:
            in_copies[ci + _NBUF] = start_in(ci + _NBUF)

    for ci in range(max(0, _NCHUNK - _NBUF), _NCHUNK):
        out_copies[ci].wait()


def kernel(input, val_table):
    xt = input.T
    mesh = plsc.VectorSubcoreMesh(core_axis_name="c", subcore_axis_name="s")
    run = pl.kernel(
        _body,
        mesh=mesh,
        out_type=jax.ShapeDtypeStruct((_ROWS, _COLS), jnp.float32),
        scratch_types=[
            pltpu.VMEM((_ORDER,), jnp.float32),
            pltpu.VMEM((_NBUF, _CR, _CW), jnp.int32),
            pltpu.VMEM((_NBUF, _CR, _CW), jnp.float32),
            pltpu.SemaphoreType.DMA((_NBUF,)),
            pltpu.SemaphoreType.DMA((_NBUF,)),
        ],
        compiler_params=pltpu.CompilerParams(
            needs_layout_passes=False, use_tc_tiling_on_sc=True,
            skip_device_barrier=True),
    )
    return run(xt, val_table).T


# speedup vs baseline: 1.0415x; 1.0035x over previous
import functools

import jax
import jax.numpy as jnp
from jax import lax
from jax.experimental import pallas as pl
from jax.experimental.pallas import tpu as pltpu
from jax.experimental.pallas import tpu_sc as plsc

_ORDER = 16
_L = 16
_NC = 2
_NS = 16
_NW = _NC * _NS
_ROWS = 200
_COLS = 16384
_CW = _COLS // _NW
_CR = 40
_CHUNK_ROWS = (8, 40, 40, 40, 40, 24, 8)
_CHUNK_OFFS = (0, 8, 48, 88, 128, 168, 192)
_NCHUNK = len(_CHUNK_ROWS)
_CVEC = _CW // _L
_NBUF = 3


def _body(inp_hbm, table_hbm, out_hbm, table_v, ins, outs, sin, sout):
    wid = lax.axis_index("s") * _NC + lax.axis_index("c")
    col0 = wid * _CW

    def start_in(ci):
        b = ci % _NBUF
        r0, nr = _CHUNK_OFFS[ci], _CHUNK_ROWS[ci]
        return pltpu.async_copy(
            inp_hbm.at[pl.ds(r0, nr), pl.ds(col0, _CW)],
            ins.at[b, pl.ds(0, nr)], sin.at[b])

    def start_out(ci):
        b = ci % _NBUF
        r0, nr = _CHUNK_OFFS[ci], _CHUNK_ROWS[ci]
        return pltpu.async_copy(
            outs.at[b, pl.ds(0, nr)],
            out_hbm.at[pl.ds(r0, nr), pl.ds(col0, _CW)], sout.at[b])

    in_copies = {ci: start_in(ci) for ci in range(min(_NBUF, _NCHUNK))}
    out_copies = {}
    pltpu.sync_copy(table_hbm, table_v)
    tbl = table_v[...]
    for ci in range(_NCHUNK):
        b = ci % _NBUF
        in_copies[ci].wait()
        if ci >= _NBUF:
            out_copies[ci - _NBUF].wait()
        iv, ov = ins.at[b], outs.at[b]

        @plsc.parallel_loop(0, _CHUNK_ROWS[ci] * _CVEC, unroll=16)
        def _gather(i, iv=iv, ov=ov):
            r = lax.shift_right_logical(i, 5)
            c = lax.shift_left(lax.bitwise_and(i, _CVEC - 1), 4)
            idx = iv[r, pl.ds(c, _L)]
            ov[r, pl.ds(c, _L)] = jnp.take_along_axis(
                tbl, idx, axis=0, mode="promise_in_bounds")

        out_copies[ci] = start_out(ci)
        if ci + _NBUF < _NCHUNK:
            in_copies[ci + _NBUF] = start_in(ci + _NBUF)

    for ci in range(max(0, _NCHUNK - _NBUF), _NCHUNK):
        out_copies[ci].wait()


def kernel(input, val_table):
    xt = input.T
    mesh = plsc.VectorSubcoreMesh(core_axis_name="c", subcore_axis_name="s")
    run = pl.kernel(
        _body,
        mesh=mesh,
        out_type=jax.ShapeDtypeStruct((_ROWS, _COLS), jnp.float32),
        scratch_types=[
            pltpu.VMEM((_ORDER,), jnp.float32),
            pltpu.VMEM((_NBUF, _CR, _CW), jnp.int32),
            pltpu.VMEM((_NBUF, _CR, _CW), jnp.float32),
            pltpu.SemaphoreType.DMA((_NBUF,)),
            pltpu.SemaphoreType.DMA((_NBUF,)),
        ],
        compiler_params=pltpu.CompilerParams(
            needs_layout_passes=False, use_tc_tiling_on_sc=True,
            skip_device_barrier=True),
    )
    return run(xt, val_table).T
